# Initial kernel scaffold; baseline (speedup 1.0000x reference)
import functools

import jax
import jax.numpy as jnp
from jax import lax
from jax.experimental import pallas as pl
from jax.experimental.pallas import tpu as pltpu
from jax.experimental.pallas import tpu_sc as plsc

BATCH = 16384
HIST = 200
EMBED = 32
DUR_PAD = 208

NUM_CORES = 2
NUM_SUBCORES = 16
NUM_WORKERS = NUM_CORES * NUM_SUBCORES
ROWS_PER_WORKER = BATCH // NUM_WORKERS


def _build_gather_add():
  mesh = plsc.VectorSubcoreMesh(core_axis_name="c", subcore_axis_name="s")

  @functools.partial(
      pl.kernel,
      mesh=mesh,
      out_type=jax.ShapeDtypeStruct((BATCH, HIST, EMBED), jnp.float32),
      scratch_types=[
          pltpu.VMEM((2, 128), jnp.int32),
          pltpu.VMEM((HIST, EMBED), jnp.float32),
          pltpu.VMEM((DUR_PAD,), jnp.float32),
          pltpu.SemaphoreType.DMA,
      ],
  )
  def gather_add(table_hbm, idx_hbm, dur_hbm, out_hbm, idx_v, rows_v,
                 dur_v, sem):
    wid = lax.axis_index("s") * NUM_CORES + lax.axis_index("c")
    base = wid * ROWS_PER_WORKER

    def row_body(b, carry):
      pltpu.sync_copy(idx_hbm.at[b, pl.ds(0, 128)], idx_v.at[0])
      pltpu.sync_copy(idx_hbm.at[b, pl.ds(128, 72)],
                      idx_v.at[1, pl.ds(0, 72)])
      cp0 = pltpu.async_copy(table_hbm.at[idx_v.at[0]],
                             rows_v.at[pl.ds(0, 128)], sem)
      cp1 = pltpu.async_copy(table_hbm.at[idx_v.at[1, pl.ds(0, 72)]],
                             rows_v.at[pl.ds(128, 72)], sem)
      pltpu.sync_copy(dur_hbm.at[b], dur_v)
      cp0.wait()
      cp1.wait()

      acc = jnp.zeros((16,), jnp.float32)
      for j in range(DUR_PAD // 16):
        acc = acc + dur_v[pl.ds(j * 16, 16)]
      mean_vec = jnp.full((16,), jnp.sum(acc) * (1.0 / HIST))

      def add_body(r, mv):
        dv = plsc.load_gather(dur_v, [jnp.full((16,), r, jnp.int32)]) - mv
        rows_v[r, pl.ds(0, 16)] = rows_v[r, pl.ds(0, 16)] + dv
        rows_v[r, pl.ds(16, 16)] = rows_v[r, pl.ds(16, 16)] + dv
        return mv

      lax.fori_loop(0, HIST, add_body, mean_vec, unroll=8)
      pltpu.sync_copy(rows_v, out_hbm.at[b])
      return carry

    lax.fori_loop(base, base + ROWS_PER_WORKER, row_body, 0)

  return gather_add


_gather_add = _build_gather_add()


def kernel(x, table):
  idx = x[..., 0].astype(jnp.int32)
  dur = x[..., 1]
  dur_p = jnp.pad(dur, ((0, 0), (0, DUR_PAD - HIST)))
  return _gather_add(table, idx, dur_p)


# speedup vs baseline: 3.0607x; 3.0607x over previous
import functools

import jax
import jax.numpy as jnp
from jax import lax
from jax.experimental import pallas as pl
from jax.experimental.pallas import tpu as pltpu
from jax.experimental.pallas import tpu_sc as plsc

BATCH = 16384
HIST = 200
EMBED = 32
DUR_PAD = 208

NUM_CORES = 2
NUM_SUBCORES = 16
NUM_WORKERS = NUM_CORES * NUM_SUBCORES
ROWS_PER_WORKER = BATCH // NUM_WORKERS


def _build_gather_add():
  mesh = plsc.VectorSubcoreMesh(core_axis_name="c", subcore_axis_name="s")

  @functools.partial(
      pl.kernel,
      mesh=mesh,
      out_type=jax.ShapeDtypeStruct((BATCH, HIST, EMBED), jnp.float32),
      compiler_params=pltpu.CompilerParams(
          needs_layout_passes=False, use_tc_tiling_on_sc=False),
      scratch_types=[
          pltpu.VMEM((2, 128), jnp.int32),
          pltpu.VMEM((HIST, EMBED), jnp.float32),
          pltpu.VMEM((DUR_PAD,), jnp.float32),
          pltpu.SemaphoreType.DMA,
      ],
  )
  def gather_add(table_hbm, idx_hbm, dur_hbm, out_hbm, idx_v, rows_v,
                 dur_v, sem):
    wid = lax.axis_index("s") * NUM_CORES + lax.axis_index("c")
    base = wid * ROWS_PER_WORKER

    def row_body(b, carry):
      pltpu.sync_copy(idx_hbm.at[b, pl.ds(0, 128)], idx_v.at[0])
      pltpu.sync_copy(idx_hbm.at[b, pl.ds(128, 72)],
                      idx_v.at[1, pl.ds(0, 72)])
      cp0 = pltpu.async_copy(table_hbm.at[idx_v.at[0]],
                             rows_v.at[pl.ds(0, 128)], sem)
      cp1 = pltpu.async_copy(table_hbm.at[idx_v.at[1, pl.ds(0, 72)]],
                             rows_v.at[pl.ds(128, 72)], sem)
      pltpu.sync_copy(dur_hbm.at[b], dur_v)
      cp0.wait()
      cp1.wait()

      acc = jnp.zeros((16,), jnp.float32)
      for j in range(DUR_PAD // 16):
        acc = acc + dur_v[pl.ds(j * 16, 16)]
      lane = lax.iota(jnp.int32, 16)
      dnums = lax.GatherDimensionNumbers(
          offset_dims=(), collapsed_slice_dims=(0,), start_index_map=(0,))
      for sh in (1, 2, 4, 8):
        perm = (lane ^ sh).reshape(16, 1)
        acc = acc + lax.gather(
            acc, perm, dnums, (1,),
            mode=lax.GatherScatterMode.PROMISE_IN_BOUNDS)
      mean_vec = acc * (1.0 / HIST)

      def add_body(r, mv):
        dv = plsc.load_gather(dur_v, [jnp.full((16,), r, jnp.int32)]) - mv
        rows_v[r, pl.ds(0, 16)] = rows_v[r, pl.ds(0, 16)] + dv
        rows_v[r, pl.ds(16, 16)] = rows_v[r, pl.ds(16, 16)] + dv
        return mv

      lax.fori_loop(0, HIST, add_body, mean_vec, unroll=8)
      pltpu.sync_copy(rows_v, out_hbm.at[b])
      return carry

    lax.fori_loop(base, base + ROWS_PER_WORKER, row_body, 0)

  return gather_add


_gather_add = _build_gather_add()


def kernel(x, table):
  idx = x[..., 0].astype(jnp.int32)
  dur = x[..., 1]
  dur_p = jnp.pad(dur, ((0, 0), (0, DUR_PAD - HIST)))
  return _gather_add(table, idx, dur_p)


# speedup vs baseline: 4.1003x; 1.3397x over previous
import functools

import jax
import jax.numpy as jnp
from jax import lax
from jax.experimental import pallas as pl
from jax.experimental.pallas import tpu as pltpu
from jax.experimental.pallas import tpu_sc as plsc

BATCH = 16384
HIST = 200
EMBED = 32
DUR_PAD = 208

NUM_CORES = 2
NUM_SUBCORES = 16
NUM_WORKERS = NUM_CORES * NUM_SUBCORES
ROWS_PER_WORKER = BATCH // NUM_WORKERS
ELEMS_PER_WORKER = ROWS_PER_WORKER * HIST
GROUP = 128
GROUPS_PER_WORKER = ELEMS_PER_WORKER // GROUP
NGROUPS_TOTAL = BATCH * HIST // GROUP
NB = 4
DURA_CHUNK = 16


def _build_gather_add():
  mesh = plsc.VectorSubcoreMesh(core_axis_name="c", subcore_axis_name="s")

  @functools.partial(
      pl.kernel,
      mesh=mesh,
      out_type=jax.ShapeDtypeStruct((NGROUPS_TOTAL, GROUP, EMBED),
                                    jnp.float32),
      compiler_params=pltpu.CompilerParams(
          needs_layout_passes=False, use_tc_tiling_on_sc=False),
      scratch_types=[
          pltpu.VMEM((ELEMS_PER_WORKER + 16,), jnp.float32),
          pltpu.VMEM((DURA_CHUNK, DUR_PAD), jnp.float32),
          pltpu.VMEM((NB, GROUP), jnp.int32),
          pltpu.VMEM((NB, GROUP, EMBED), jnp.float32),
          pltpu.SemaphoreType.DMA((NB,)),
          pltpu.SemaphoreType.DMA((NB,)),
          pltpu.SemaphoreType.DMA((NB,)),
      ],
  )
  def gather_add(table_hbm, idx_hbm, dur_hbm, out_hbm, durc_v, dur_buf,
                 idx_buf, rows_buf, idx_sem, gather_sem, out_sem):
    wid = lax.axis_index("s") * NUM_CORES + lax.axis_index("c")
    base_row = wid * ROWS_PER_WORKER
    base_group = wid * GROUPS_PER_WORKER

    for b in range(NB):
      pltpu.async_copy(idx_hbm.at[base_group + b], idx_buf.at[b],
                       idx_sem.at[b])
    pltpu.make_async_copy(idx_hbm.at[base_group], idx_buf.at[0],
                          idx_sem.at[0]).wait()
    pltpu.async_copy(table_hbm.at[idx_buf.at[0]], rows_buf.at[0],
                     gather_sem.at[0])

    lane = lax.iota(jnp.int32, 16)
    dnums = lax.GatherDimensionNumbers(
        offset_dims=(), collapsed_slice_dims=(0,), start_index_map=(0,))

    def chunk_a(ci, carry):
      row0 = base_row + ci * DURA_CHUNK
      pltpu.sync_copy(dur_hbm.at[pl.ds(row0, DURA_CHUNK)], dur_buf)

      def row_a(j, c2):
        acc = jnp.zeros((16,), jnp.float32)
        for k in range(DUR_PAD // 16):
          acc = acc + dur_buf[j, pl.ds(k * 16, 16)]
        for sh in (1, 2, 4, 8):
          perm = (lane ^ sh).reshape(16, 1)
          acc = acc + lax.gather(
              acc, perm, dnums, (1,),
              mode=lax.GatherScatterMode.PROMISE_IN_BOUNDS)
        mean_vec = acc * (1.0 / HIST)
        off = (ci * DURA_CHUNK + j) * HIST
        for k in range(DUR_PAD // 16):
          durc_v[pl.ds(off + k * 16, 16)] = (
              dur_buf[j, pl.ds(k * 16, 16)] - mean_vec)
        return c2

      lax.fori_loop(0, DURA_CHUNK, row_a, 0)
      return carry

    lax.fori_loop(0, ROWS_PER_WORKER // DURA_CHUNK, chunk_a, 0)

    def pipe(m, carry):
      for b in range(NB):
        g = m * NB + b
        bn = (b + 1) % NB

        @pl.when(g < GROUPS_PER_WORKER - 1)
        def _():
          pltpu.make_async_copy(idx_hbm.at[base_group + g + 1],
                                idx_buf.at[bn], idx_sem.at[bn]).wait()

          @pl.when(g >= NB - 1)
          def _():
            pltpu.make_async_copy(
                rows_buf.at[bn],
                out_hbm.at[base_group + g + 1 - NB],
                out_sem.at[bn]).wait()

          pltpu.async_copy(table_hbm.at[idx_buf.at[bn]], rows_buf.at[bn],
                           gather_sem.at[bn])

        pltpu.make_async_copy(table_hbm.at[idx_buf.at[b]], rows_buf.at[b],
                              gather_sem.at[b]).wait()

        @pl.when(g + NB < GROUPS_PER_WORKER)
        def _():
          pltpu.async_copy(idx_hbm.at[base_group + g + NB], idx_buf.at[b],
                           idx_sem.at[b])

        rb = rows_buf.at[b]
        dbase = g * GROUP

        def add_row(r, c2):
          iv = jnp.full((16,), dbase + r, jnp.int32)
          av = plsc.load_gather(durc_v, [iv])
          rb[r, pl.ds(0, 16)] = rb[r, pl.ds(0, 16)] + av
          rb[r, pl.ds(16, 16)] = rb[r, pl.ds(16, 16)] + av
          return c2

        lax.fori_loop(0, GROUP, add_row, 0, unroll=16)
        pltpu.async_copy(rb, out_hbm.at[base_group + g], out_sem.at[b])
      return carry

    lax.fori_loop(0, GROUPS_PER_WORKER // NB, pipe, 0)

    for b in range(NB):
      pltpu.make_async_copy(
          rows_buf.at[b],
          out_hbm.at[base_group + GROUPS_PER_WORKER - NB + b],
          out_sem.at[b]).wait()

  return gather_add


_gather_add = _build_gather_add()


def kernel(x, table):
  idx = x[..., 0].astype(jnp.int32).reshape(NGROUPS_TOTAL, GROUP)
  dur = x[..., 1]
  dur_p = jnp.pad(dur, ((0, 0), (0, DUR_PAD - HIST)))
  out = _gather_add(table, idx, dur_p)
  return out.reshape(BATCH, HIST, EMBED)
